# Initial kernel scaffold; baseline (speedup 1.0000x reference)
import jax
import jax.numpy as jnp
from jax.experimental import pallas as pl


def kernel(pos_u, pos_v, neg_v, u_weight, v_weight):
    raise NotImplementedError("write your pallas kernel here")


# speedup vs baseline: 4.9127x; 4.9127x over previous
import functools

import jax
import jax.numpy as jnp
from jax import lax
from jax.experimental import pallas as pl
from jax.experimental.pallas import tpu as pltpu
from jax.experimental.pallas import tpu_sc as plsc

_VOCAB = 1000000
_DIM = 64
_B = 16384
_NNEG = 20

_NC = 2
_NS = 16
_NW = _NC * _NS
_CB = _B // _NW
_E = 32
_NCHUNK = _CB // _E
_NROW = _E * _NNEG // 128


def _sc_scores(pu, pv, ng, u_weight, v_weight):
    mesh = plsc.VectorSubcoreMesh(core_axis_name="c", subcore_axis_name="s")

    @functools.partial(
        pl.kernel,
        out_type=[
            jax.ShapeDtypeStruct((_NW, _NCHUNK, _E), jnp.float32),
            jax.ShapeDtypeStruct((_NW, _NCHUNK, _E * _NNEG), jnp.float32),
        ],
        mesh=mesh,
        compiler_params=pltpu.CompilerParams(
            needs_layout_passes=False, use_tc_tiling_on_sc=False),
        scratch_types=[
            pltpu.VMEM((_NCHUNK, _E), jnp.int32),
            pltpu.VMEM((_NCHUNK, _E), jnp.int32),
            pltpu.VMEM((_NCHUNK, _NROW, 128), jnp.int32),
            pltpu.VMEM((2, _E, _DIM), jnp.float32),
            pltpu.VMEM((2, _E, _DIM), jnp.float32),
            pltpu.VMEM((2, _NROW, 128, _DIM), jnp.float32),
            pltpu.VMEM((_E,), jnp.float32),
            pltpu.VMEM((_E * _NNEG,), jnp.float32),
            pltpu.SemaphoreType.DMA,
            pltpu.SemaphoreType.DMA,
        ],
    )
    def k(pu_hbm, pv_hbm, ng_hbm, uw_hbm, vw_hbm, pos_out, neg_out,
          uidx, vidx, nidx, ubuf, vbuf, nbuf, pbuf, sbuf, sem0, sem1):
        w = lax.axis_index("s") * _NC + lax.axis_index("c")
        pltpu.sync_copy(pu_hbm.at[w], uidx)
        pltpu.sync_copy(pv_hbm.at[w], vidx)
        pltpu.sync_copy(ng_hbm.at[w], nidx)

        sems = (sem0, sem1)

        def fire(c):
            pb = c % 2
            sem = sems[pb]
            d = [
                pltpu.async_copy(uw_hbm.at[uidx.at[c]], ubuf.at[pb], sem),
                pltpu.async_copy(vw_hbm.at[vidx.at[c]], vbuf.at[pb], sem),
            ]
            for j in range(_NROW):
                d.append(
                    pltpu.async_copy(vw_hbm.at[nidx.at[c, j]], nbuf.at[pb, j],
                                     sem))
            return d

        lane15 = lax.iota(jnp.int32, 16) == 15

        def put(ref1d, pos, acc):
            s = plsc.cumsum(acc)
            plsc.store_scatter(
                ref1d, [jnp.full((16,), pos, jnp.int32)], s, mask=lane15)

        pend = fire(0)
        for c in range(_NCHUNK):
            nxt = fire(c + 1) if c + 1 < _NCHUNK else None
            for dsc in pend:
                dsc.wait()
            pb = c % 2

            def body(e, _, pb=pb):
                u = [ubuf[pb, e, pl.ds(16 * t, 16)] for t in range(4)]
                acc = u[0] * vbuf[pb, e, pl.ds(0, 16)]
                for t in range(1, 4):
                    acc += u[t] * vbuf[pb, e, pl.ds(16 * t, 16)]
                put(pbuf, e, acc)
                r0 = e * _NNEG
                for j in range(_NNEG):
                    r = r0 + j
                    row = r >> 7
                    col = r & 127
                    acc = u[0] * nbuf[pb, row, col, pl.ds(0, 16)]
                    for t in range(1, 4):
                        acc += u[t] * nbuf[pb, row, col, pl.ds(16 * t, 16)]
                    put(sbuf, r, acc)
                return 0

            lax.fori_loop(0, _E, body, 0)
            pltpu.sync_copy(pbuf, pos_out.at[w, c])
            pltpu.sync_copy(sbuf, neg_out.at[w, c])
            pend = nxt

    return k(pu, pv, ng, u_weight, v_weight)


def _tc_loss(pos2, neg2):
    def body(p_ref, n_ref, loss_ref, corr_ref):
        p = p_ref[...]
        n = n_ref[...]

        def logsig(t):
            return jnp.minimum(t, 0.0) - jnp.log1p(jnp.exp(-jnp.abs(t)))

        loss_ref[0, 0] = -(jnp.sum(logsig(p)) + jnp.sum(logsig(-n)))
        corr_ref[0, 0] = jnp.sum((p > 0.0).astype(jnp.int32))

    loss, corr = pl.pallas_call(
        body,
        out_shape=[
            jax.ShapeDtypeStruct((1, 1), jnp.float32),
            jax.ShapeDtypeStruct((1, 1), jnp.int32),
        ],
        out_specs=[
            pl.BlockSpec(memory_space=pltpu.SMEM),
            pl.BlockSpec(memory_space=pltpu.SMEM),
        ],
    )(pos2, neg2)
    return loss[0, 0], corr[0, 0]


def kernel(pos_u, pos_v, neg_v, u_weight, v_weight):
    pu = pos_u.reshape(_NW, _NCHUNK, _E)
    pv = pos_v.reshape(_NW, _NCHUNK, _E)
    ng = neg_v.reshape(_NW, _NCHUNK, _NROW, 128)
    pos_s, neg_s = _sc_scores(pu, pv, ng, u_weight, v_weight)
    loss, corr = _tc_loss(
        pos_s.reshape(_B // 128, 128),
        neg_s.reshape(_B * _NNEG // 128, 128),
    )
    total = jnp.array(_B, dtype=jnp.int32)
    return (loss, corr, total)


# speedup vs baseline: 5.2150x; 1.0615x over previous
import functools

import jax
import jax.numpy as jnp
from jax import lax
from jax.experimental import pallas as pl
from jax.experimental.pallas import tpu as pltpu
from jax.experimental.pallas import tpu_sc as plsc

_VOCAB = 1000000
_DIM = 64
_B = 16384
_NNEG = 20

_NC = 2
_NS = 16
_NW = _NC * _NS
_CB = _B // _NW
_E = 16
_NCHUNK = _CB // _E
_NPC = _E * _NNEG
_NIW = 80
_NIR = _NPC // _NIW


def _sc_scores(pu, pv, ng, uw128, vw128):
    mesh = plsc.VectorSubcoreMesh(core_axis_name="c", subcore_axis_name="s")

    @functools.partial(
        pl.kernel,
        out_type=[
            jax.ShapeDtypeStruct((_NW, _NCHUNK, _E), jnp.float32),
            jax.ShapeDtypeStruct((_NW, _NCHUNK, _NPC), jnp.float32),
        ],
        mesh=mesh,
        compiler_params=pltpu.CompilerParams(
            needs_layout_passes=False, use_tc_tiling_on_sc=False),
        scratch_types=[
            pltpu.VMEM((_NCHUNK, _E), jnp.int32),
            pltpu.VMEM((_NCHUNK, _E), jnp.int32),
            pltpu.VMEM((_NCHUNK, _NIR, _NIW), jnp.int32),
            pltpu.VMEM((2, _E, 128), jnp.float32),
            pltpu.VMEM((2, _E, 128), jnp.float32),
            pltpu.VMEM((2, _NPC, 128), jnp.float32),
            pltpu.VMEM((_E,), jnp.float32),
            pltpu.VMEM((_NPC,), jnp.float32),
            pltpu.SemaphoreType.DMA,
            pltpu.SemaphoreType.DMA,
        ],
    )
    def k(pu_hbm, pv_hbm, ng_hbm, uw_hbm, vw_hbm, pos_out, neg_out,
          uidx, vidx, nidx, ubuf, vbuf, nbuf, pbuf, sbuf, sem0, sem1):
        w = lax.axis_index("s") * _NC + lax.axis_index("c")
        pltpu.sync_copy(pu_hbm.at[w], uidx)
        pltpu.sync_copy(pv_hbm.at[w], vidx)
        pltpu.sync_copy(ng_hbm.at[w], nidx)

        def copies(c, pb, sem, make):
            yield make(uw_hbm.at[uidx.at[c]], ubuf.at[pb], sem)
            yield make(vw_hbm.at[vidx.at[c]], vbuf.at[pb], sem)
            for j in range(_NIR):
                yield make(vw_hbm.at[nidx.at[c, j]],
                           nbuf.at[pb, pl.ds(j * _NIW, _NIW)], sem)

        def fire(c, pb, sem):
            for _ in copies(c, pb, sem, pltpu.async_copy):
                pass

        def drain(c, pb, sem):
            for d in copies(c, pb, sem, pltpu.make_async_copy):
                d.wait()

        lane15 = lax.iota(jnp.int32, 16) == 15

        def put(ref1d, pos, acc):
            s = plsc.cumsum(acc)
            plsc.store_scatter(
                ref1d, [jnp.full((16,), pos, jnp.int32)], s, mask=lane15)

        def compute(c, pb):
            def body(e, _):
                u = [ubuf[pb, e, pl.ds(16 * t, 16)] for t in range(4)]
                acc = u[0] * vbuf[pb, e, pl.ds(0, 16)]
                for t in range(1, 4):
                    acc += u[t] * vbuf[pb, e, pl.ds(16 * t, 16)]
                put(pbuf, e, acc)
                r0 = e * _NNEG
                for j in range(_NNEG):
                    r = r0 + j
                    acc = u[0] * nbuf[pb, r, pl.ds(0, 16)]
                    for t in range(1, 4):
                        acc += u[t] * nbuf[pb, r, pl.ds(16 * t, 16)]
                    put(sbuf, r, acc)
                return 0

            lax.fori_loop(0, _E, body, 0)
            pltpu.sync_copy(pbuf, pos_out.at[w, c])
            pltpu.sync_copy(sbuf, neg_out.at[w, c])

        fire(0, 0, sem0)
        fire(1, 1, sem1)

        def pair(i, _):
            c = 2 * i
            drain(c, 0, sem0)
            compute(c, 0)

            @pl.when(c + 2 < _NCHUNK)
            def _():
                fire(c + 2, 0, sem0)

            drain(c + 1, 1, sem1)
            compute(c + 1, 1)

            @pl.when(c + 3 < _NCHUNK)
            def _():
                fire(c + 3, 1, sem1)

            return 0

        lax.fori_loop(0, _NCHUNK // 2, pair, 0)

    return k(pu, pv, ng, uw128, vw128)


def _tc_loss(pos2, neg2):
    def body(p_ref, n_ref, loss_ref, corr_ref):
        p = p_ref[...]
        n = n_ref[...]

        def logsig(t):
            return jnp.minimum(t, 0.0) - jnp.log1p(jnp.exp(-jnp.abs(t)))

        loss_ref[0, 0] = -(jnp.sum(logsig(p)) + jnp.sum(logsig(-n)))
        corr_ref[0, 0] = jnp.sum((p > 0.0).astype(jnp.int32))

    loss, corr = pl.pallas_call(
        body,
        out_shape=[
            jax.ShapeDtypeStruct((1, 1), jnp.float32),
            jax.ShapeDtypeStruct((1, 1), jnp.int32),
        ],
        out_specs=[
            pl.BlockSpec(memory_space=pltpu.SMEM),
            pl.BlockSpec(memory_space=pltpu.SMEM),
        ],
    )(pos2, neg2)
    return loss[0, 0], corr[0, 0]


def kernel(pos_u, pos_v, neg_v, u_weight, v_weight):
    uw128 = jnp.pad(u_weight, ((0, 0), (0, 128 - _DIM)))
    vw128 = jnp.pad(v_weight, ((0, 0), (0, 128 - _DIM)))
    pu = pos_u.reshape(_NW, _NCHUNK, _E)
    pv = pos_v.reshape(_NW, _NCHUNK, _E)
    ng = neg_v.reshape(_NW, _NCHUNK, _NIR, _NIW)
    pos_s, neg_s = _sc_scores(pu, pv, ng, uw128, vw128)
    loss, corr = _tc_loss(
        pos_s.reshape(_B // 128, 128),
        neg_s.reshape(_B * _NNEG // 128, 128),
    )
    total = jnp.array(_B, dtype=jnp.int32)
    return (loss, corr, total)


# speedup vs baseline: 5.3546x; 1.0268x over previous
import functools

import jax
import jax.numpy as jnp
from jax import lax
from jax.experimental import pallas as pl
from jax.experimental.pallas import tpu as pltpu
from jax.experimental.pallas import tpu_sc as plsc

_VOCAB = 1000000
_DIM = 64
_B = 16384
_NNEG = 20

_NC = 2
_NS = 16
_NW = _NC * _NS
_CB = _B // _NW
_E = 16
_NCHUNK = _CB // _E
_NPC = _E * _NNEG
_NIW = 80
_NIR = _NPC // _NIW


def _sc_scores(pu, pv, ng, uw128, vw128):
    mesh = plsc.VectorSubcoreMesh(core_axis_name="c", subcore_axis_name="s")

    @functools.partial(
        pl.kernel,
        out_type=[
            jax.ShapeDtypeStruct((_NW, _NCHUNK, _E), jnp.float32),
            jax.ShapeDtypeStruct((_NW, _NCHUNK, _NPC), jnp.float32),
        ],
        mesh=mesh,
        compiler_params=pltpu.CompilerParams(
            needs_layout_passes=False, use_tc_tiling_on_sc=False),
        scratch_types=[
            pltpu.VMEM((_NCHUNK, _E), jnp.int32),
            pltpu.VMEM((_NCHUNK, _E), jnp.int32),
            pltpu.VMEM((_NCHUNK, _NIR, _NIW), jnp.int32),
            pltpu.VMEM((2, _E, 128), jnp.float32),
            pltpu.VMEM((2, _E, 128), jnp.float32),
            pltpu.VMEM((2, _NPC, 128), jnp.float32),
            pltpu.VMEM((_E,), jnp.float32),
            pltpu.VMEM((_NPC,), jnp.float32),
            pltpu.SemaphoreType.DMA,
            pltpu.SemaphoreType.DMA,
        ],
    )
    def k(pu_hbm, pv_hbm, ng_hbm, uw_hbm, vw_hbm, pos_out, neg_out,
          uidx, vidx, nidx, ubuf, vbuf, nbuf, pbuf, sbuf, sem0, sem1):
        w = lax.axis_index("s") * _NC + lax.axis_index("c")
        pltpu.sync_copy(pu_hbm.at[w], uidx)
        pltpu.sync_copy(pv_hbm.at[w], vidx)
        pltpu.sync_copy(ng_hbm.at[w], nidx)

        def copies(c, pb, sem, make):
            yield make(uw_hbm.at[uidx.at[c]], ubuf.at[pb], sem)
            yield make(vw_hbm.at[vidx.at[c]], vbuf.at[pb], sem)
            for j in range(_NIR):
                yield make(vw_hbm.at[nidx.at[c, j]],
                           nbuf.at[pb, pl.ds(j * _NIW, _NIW)], sem)

        def fire(c, pb, sem):
            for _ in copies(c, pb, sem, pltpu.async_copy):
                pass

        def drain(c, pb, sem):
            for d in copies(c, pb, sem, pltpu.make_async_copy):
                d.wait()

        lane15 = lax.iota(jnp.int32, 16) == 15

        def put(ref1d, pos, acc):
            s = plsc.cumsum(acc)
            plsc.store_scatter(
                ref1d, [jnp.full((16,), pos, jnp.int32)], s, mask=lane15)

        def compute(c, pb):
            def body(e, _):
                u = [ubuf[pb, e, pl.ds(16 * t, 16)] for t in range(4)]
                acc = u[0] * vbuf[pb, e, pl.ds(0, 16)]
                for t in range(1, 4):
                    acc += u[t] * vbuf[pb, e, pl.ds(16 * t, 16)]
                put(pbuf, e, acc)
                r0 = e * _NNEG
                for j in range(_NNEG):
                    r = r0 + j
                    acc = u[0] * nbuf[pb, r, pl.ds(0, 16)]
                    for t in range(1, 4):
                        acc += u[t] * nbuf[pb, r, pl.ds(16 * t, 16)]
                    put(sbuf, r, acc)
                return 0

            lax.fori_loop(0, _E, body, 0)
            pltpu.sync_copy(pbuf, pos_out.at[w, c])
            pltpu.sync_copy(sbuf, neg_out.at[w, c])

        fire(0, 0, sem0)
        fire(1, 1, sem1)

        def pair(i, _):
            c = 2 * i
            drain(c, 0, sem0)
            compute(c, 0)

            @pl.when(c + 2 < _NCHUNK)
            def _():
                fire(c + 2, 0, sem0)

            drain(c + 1, 1, sem1)
            compute(c + 1, 1)

            @pl.when(c + 3 < _NCHUNK)
            def _():
                fire(c + 3, 1, sem1)

            return 0

        lax.fori_loop(0, _NCHUNK // 2, pair, 0)

    return k(pu, pv, ng, uw128, vw128)


_VB = 2048


def _repack(table):
    ut = jnp.swapaxes(table, 0, 1)

    def body(ut_ref, out_ref):
        x = ut_ref[...]
        r = lax.broadcasted_iota(jnp.int32, (_DIM, 128), 0)
        c = lax.broadcasted_iota(jnp.int32, (_DIM, 128), 1)
        w = (r == c % _DIM).astype(jnp.float32)
        out_ref[...] = lax.dot_general(
            x, w, (((0,), (0,)), ((), ())),
            preferred_element_type=jnp.float32)

    return pl.pallas_call(
        body,
        grid=((_VOCAB + _VB - 1) // _VB,),
        in_specs=[pl.BlockSpec((_DIM, _VB), lambda i: (0, i))],
        out_specs=pl.BlockSpec((_VB, 128), lambda i: (i, 0)),
        out_shape=jax.ShapeDtypeStruct((_VOCAB, 128), jnp.float32),
    )(ut)


def _tc_loss(pos2, neg2):
    def body(p_ref, n_ref, loss_ref, corr_ref):
        p = p_ref[...]
        n = n_ref[...]

        def logsig(t):
            return jnp.minimum(t, 0.0) - jnp.log1p(jnp.exp(-jnp.abs(t)))

        loss_ref[0, 0] = -(jnp.sum(logsig(p)) + jnp.sum(logsig(-n)))
        corr_ref[0, 0] = jnp.sum((p > 0.0).astype(jnp.int32))

    loss, corr = pl.pallas_call(
        body,
        out_shape=[
            jax.ShapeDtypeStruct((1, 1), jnp.float32),
            jax.ShapeDtypeStruct((1, 1), jnp.int32),
        ],
        out_specs=[
            pl.BlockSpec(memory_space=pltpu.SMEM),
            pl.BlockSpec(memory_space=pltpu.SMEM),
        ],
    )(pos2, neg2)
    return loss[0, 0], corr[0, 0]


def kernel(pos_u, pos_v, neg_v, u_weight, v_weight):
    uw128 = _repack(u_weight)
    vw128 = _repack(v_weight)
    pu = pos_u.reshape(_NW, _NCHUNK, _E)
    pv = pos_v.reshape(_NW, _NCHUNK, _E)
    ng = neg_v.reshape(_NW, _NCHUNK, _NIR, _NIW)
    pos_s, neg_s = _sc_scores(pu, pv, ng, uw128, vw128)
    loss, corr = _tc_loss(
        pos_s.reshape(_B // 128, 128),
        neg_s.reshape(_B * _NNEG // 128, 128),
    )
    total = jnp.array(_B, dtype=jnp.int32)
    return (loss, corr, total)
